# Initial kernel scaffold; baseline (speedup 1.0000x reference)
import jax
import jax.numpy as jnp
from jax.experimental import pallas as pl


def kernel(x, W1, g1, b1, W2, g2, b2, W3, g3, b3, W4, g4, b4, W5, g5, b5, L1, g6, b6, L2, L2b, g7, b7, L3, L3b):
    raise NotImplementedError("write your pallas kernel here")


# speedup vs baseline: 1.3057x; 1.3057x over previous
import functools

import jax
import jax.numpy as jnp
from jax import lax
from jax.experimental import pallas as pl
from jax.experimental.pallas import tpu as pltpu
from jax.experimental.pallas import tpu_sc as plsc

K = 20
B = 8
N = 1024
BN = B * N
PB = 128


def _knn_body(f_ref, xxc_ref, xxr_ref, o_ref):
    F = f_ref[0]
    G = lax.dot_general(F, F, (((1,), (1,)), ((), ())),
                        preferred_element_type=jnp.float32)
    pd = (2.0 * G - xxc_ref[0]) - xxr_ref[0]
    iota = lax.broadcasted_iota(jnp.int32, (N, N), 1)
    base = pl.program_id(0) * N
    cols = []
    for _ in range(K):
        m = jnp.max(pd, axis=-1, keepdims=True)
        sel = jnp.min(jnp.where(pd == m, iota, N), axis=-1, keepdims=True)
        cols.append(sel + base)
        pd = jnp.where(iota == sel, -jnp.inf, pd)
    o_ref[0] = jnp.concatenate(cols, axis=-1)


def _mk_knn(C):
    return pl.pallas_call(
        _knn_body, grid=(B,),
        in_specs=[pl.BlockSpec((1, N, C), lambda b: (b, 0, 0)),
                  pl.BlockSpec((1, N, 1), lambda b: (b, 0, 0)),
                  pl.BlockSpec((1, 1, N), lambda b: (b, 0, 0))],
        out_specs=pl.BlockSpec((1, N, K), lambda b: (b, 0, 0)),
        out_shape=jax.ShapeDtypeStruct((B, N, K), jnp.int32))


def _ord_body(cx_ref, cy_ref, cz_ref, idx_ref, ox_ref, oy_ref, oz_ref, pi_ref):
    cxv, cyv, czv = cx_ref[...], cy_ref[...], cz_ref[...]
    idxv = idx_ref[...]
    lane = lax.broadcasted_iota(jnp.int32, (PB, K), 1)
    rank = jnp.zeros((PB, K), jnp.int32)
    for j in range(K):
        xj = cxv[:, j:j + 1]; yj = cyv[:, j:j + 1]; zj = czv[:, j:j + 1]
        l = (xj < cxv) | ((xj == cxv) & ((yj < cyv) | ((yj == cyv) &
            ((zj < czv) | ((zj == czv) & (j < lane))))))
        rank = rank + l.astype(jnp.int32)
    ox = jnp.zeros((PB, K), jnp.float32); oy = ox; oz = ox
    pi = jnp.zeros((PB, K), jnp.int32)
    for j in range(K):
        oh = (rank[:, j:j + 1] == lane)
        ohf = oh.astype(jnp.float32)
        ox = ox + ohf * cxv[:, j:j + 1]
        oy = oy + ohf * cyv[:, j:j + 1]
        oz = oz + ohf * czv[:, j:j + 1]
        pi = pi + oh.astype(jnp.int32) * idxv[:, j:j + 1]
    ox_ref[...] = ox; oy_ref[...] = oy; oz_ref[...] = oz; pi_ref[...] = pi


_ordk = pl.pallas_call(
    _ord_body, grid=(BN // PB,),
    in_specs=[pl.BlockSpec((PB, K), lambda i: (i, 0))] * 4,
    out_specs=[pl.BlockSpec((PB, K), lambda i: (i, 0))] * 4,
    out_shape=[jax.ShapeDtypeStruct((BN, K), jnp.float32)] * 3 +
              [jax.ShapeDtypeStruct((BN, K), jnp.int32)])


def _gord_body(cxc, cyc, czc, cxr, cyr, czr, ox_ref, oy_ref, oz_ref):
    xj, yj, zj = cxc[0], cyc[0], czc[0]
    xi, yi, zi = cxr[0], cyr[0], czr[0]
    jlt = lax.broadcasted_iota(jnp.int32, (N, N), 0) < \
          lax.broadcasted_iota(jnp.int32, (N, N), 1)
    l = (xj < xi) | ((xj == xi) & ((yj < yi) | ((yj == yi) &
        ((zj < zi) | ((zj == zi) & jlt)))))
    rank = jnp.sum(l.astype(jnp.int32), axis=0, keepdims=True)
    oh = (rank == lax.broadcasted_iota(jnp.int32, (N, N), 0)).astype(jnp.float32)
    ox_ref[0] = jnp.sum(oh * xi, axis=-1, keepdims=True)
    oy_ref[0] = jnp.sum(oh * yi, axis=-1, keepdims=True)
    oz_ref[0] = jnp.sum(oh * zi, axis=-1, keepdims=True)


_gord = pl.pallas_call(
    _gord_body, grid=(B,),
    in_specs=[pl.BlockSpec((1, N, 1), lambda b: (b, 0, 0))] * 3 +
             [pl.BlockSpec((1, 1, N), lambda b: (b, 0, 0))] * 3,
    out_specs=[pl.BlockSpec((1, N, 1), lambda b: (b, 0, 0))] * 3,
    out_shape=[jax.ShapeDtypeStruct((B, N, 1), jnp.float32)] * 3)


def _conv_body(w_ref, a_ref, g_ref, b_ref, sc_ref, o_ref):
    r = lax.dot_general(w_ref[...], a_ref[...], (((1,), (1,)), ((), ())),
                        preferred_element_type=jnp.float32)
    v = r * sc_ref[0, 0] * g_ref[...] + b_ref[...]
    o_ref[...] = jnp.where(v >= 0, v, 0.2 * v)


def _mk_conv(O, F):
    return pl.pallas_call(
        _conv_body, grid=(16,),
        in_specs=[pl.BlockSpec((O, F), lambda i: (0, 0)),
                  pl.BlockSpec((512, F), lambda i: (i, 0)),
                  pl.BlockSpec((O, 1), lambda i: (0, 0)),
                  pl.BlockSpec((O, 1), lambda i: (0, 0)),
                  pl.BlockSpec((1, 1), lambda i: (0, 0))],
        out_specs=pl.BlockSpec((O, 512), lambda i: (0, i)),
        out_shape=jax.ShapeDtypeStruct((O, BN), jnp.float32))


def _embed_body(w_ref, a_ref, g_ref, b_ref, sc_ref, mx_ref, mn_ref):
    r = lax.dot_general(w_ref[...], a_ref[0], (((1,), (1,)), ((), ())),
                        preferred_element_type=jnp.float32)
    v = r * sc_ref[0, 0] * g_ref[...] + b_ref[...]
    v = jnp.where(v >= 0, v, 0.2 * v)
    mx_ref[0] = jnp.max(v, axis=-1, keepdims=True)
    mn_ref[0] = jnp.mean(v, axis=-1, keepdims=True)


_embed = pl.pallas_call(
    _embed_body, grid=(B,),
    in_specs=[pl.BlockSpec((1024, 512), lambda b: (0, 0)),
              pl.BlockSpec((1, N, 512), lambda b: (b, 0, 0)),
              pl.BlockSpec((1024, 1), lambda b: (0, 0)),
              pl.BlockSpec((1024, 1), lambda b: (0, 0)),
              pl.BlockSpec((1, 1), lambda b: (0, 0))],
    out_specs=[pl.BlockSpec((1, 1024, 1), lambda b: (b, 0, 0)),
               pl.BlockSpec((1, 1024, 1), lambda b: (b, 0, 0))],
    out_shape=[jax.ShapeDtypeStruct((B, 1024, 1), jnp.float32)] * 2)


def _head_body(xp_ref, l1_ref, g6_ref, b6_ref, l2_ref, l2b_ref, g7_ref,
               b7_ref, l3_ref, l3b_ref, sc_ref, o_ref):
    sc = sc_ref[0, 0]
    h = lax.dot_general(xp_ref[...], l1_ref[...], (((1,), (1,)), ((), ())),
                        preferred_element_type=jnp.float32)
    v = h * sc * g6_ref[...] + b6_ref[...]
    h = jnp.where(v >= 0, v, 0.2 * v)
    h = lax.dot_general(h, l2_ref[...], (((1,), (1,)), ((), ())),
                        preferred_element_type=jnp.float32) + l2b_ref[...]
    v = h * sc * g7_ref[...] + b7_ref[...]
    h = jnp.where(v >= 0, v, 0.2 * v)
    o_ref[...] = lax.dot_general(h, l3_ref[...], (((1,), (1,)), ((), ())),
                                 preferred_element_type=jnp.float32) + l3b_ref[...]


_head = pl.pallas_call(
    _head_body, grid=(1,),
    in_specs=[pl.BlockSpec((B, 2048), lambda i: (0, 0)),
              pl.BlockSpec((512, 2048), lambda i: (0, 0)),
              pl.BlockSpec((1, 512), lambda i: (0, 0)),
              pl.BlockSpec((1, 512), lambda i: (0, 0)),
              pl.BlockSpec((256, 512), lambda i: (0, 0)),
              pl.BlockSpec((1, 256), lambda i: (0, 0)),
              pl.BlockSpec((1, 256), lambda i: (0, 0)),
              pl.BlockSpec((1, 256), lambda i: (0, 0)),
              pl.BlockSpec((40, 256), lambda i: (0, 0)),
              pl.BlockSpec((1, 40), lambda i: (0, 0)),
              pl.BlockSpec((1, 1), lambda i: (0, 0))],
    out_specs=pl.BlockSpec((B, 40), lambda i: (0, 0)),
    out_shape=jax.ShapeDtypeStruct((B, 40), jnp.float32))


_info = plsc.get_sparse_core_info()
_NC, _NS = _info.num_cores, _info.num_subcores
_NW = _NC * _NS


def _mk_gather(D, Btot, chunk):
    nch = Btot // (_NW * chunk)
    mesh = plsc.VectorSubcoreMesh(core_axis_name="c", subcore_axis_name="s")

    @functools.partial(
        pl.kernel, mesh=mesh,
        out_type=jax.ShapeDtypeStruct((Btot, D), jnp.float32),
        scratch_types=[
            pltpu.VMEM((chunk,), jnp.int32),
            pltpu.VMEM((chunk, D), jnp.float32),
            pltpu.SemaphoreType.DMA,
        ],
        compiler_params=pltpu.CompilerParams(use_tc_tiling_on_sc=False))
    def k(table_hbm, idx_hbm, out_hbm, idx_v, rows_v, sem):
        wid = lax.axis_index("s") * _NC + lax.axis_index("c")
        for c in range(nch):
            base = (wid * nch + c) * chunk
            pltpu.sync_copy(idx_hbm.at[pl.ds(base, chunk)], idx_v)
            pltpu.async_copy(table_hbm.at[idx_v], rows_v, sem).wait()
            pltpu.sync_copy(rows_v, out_hbm.at[pl.ds(base, chunk)])
    return k


_gather16 = _mk_gather(16, BN * K, 640)
_gather64 = _mk_gather(64, BN * K, 512)
_gather128 = _mk_gather(128, BN * K, 256)


def _fix_signs(vecs):
    max_idx = jnp.argmax(jnp.abs(vecs), axis=1, keepdims=True)
    max_vals = jnp.take_along_axis(vecs, max_idx, axis=1)
    signs = jnp.sign(max_vals)
    signs = jnp.where(signs == 0, jnp.ones_like(signs), signs)
    return vecs * signs


def _so3(Rm):
    det = jnp.linalg.det(Rm)
    flip = (det < 0).astype(Rm.dtype)
    col = 1.0 - 2.0 * flip
    scale = jnp.stack([jnp.ones_like(col), jnp.ones_like(col), col], axis=-1)
    return Rm * scale[:, None, :]


def _canon(pg):
    Nk = pg.shape[1]
    centered = pg - jnp.mean(pg, axis=1, keepdims=True)
    cov = jnp.einsum('mki,mkj->mij', centered, centered) / (Nk - 1)
    _, vecs = jnp.linalg.eigh(cov)
    vecs = vecs[:, :, ::-1]
    vecs = _fix_signs(vecs)
    vecs = _so3(vecs)
    cp = jnp.einsum('mki,mij->mkj', centered, vecs)
    skew = jnp.mean(cp ** 3, axis=1)
    s = jnp.sign(skew)
    s = jnp.where(s == 0, jnp.ones_like(s), s)
    odd = (jnp.sum((s < 0).astype(jnp.int32), axis=-1) % 2) == 1
    fix = jnp.stack([jnp.ones(odd.shape, s.dtype), jnp.ones(odd.shape, s.dtype),
                     jnp.where(odd, -1.0, 1.0).astype(s.dtype)], axis=-1)
    return cp * (s * fix)[:, None, :]


def _stage(featT, ptstab, feattab, gatherF, W, g, b, sc, first, O, C):
    xxr = jnp.sum(featT ** 2, axis=1, keepdims=True)
    xxc = jnp.transpose(xxr, (0, 2, 1))
    featN = jnp.transpose(featT, (0, 2, 1))
    idx = _mk_knn(C)(featN, xxc, xxr)
    idxg = idx.reshape(BN, K)
    pg = _gather16(ptstab, idxg.reshape(-1))
    cp2 = _canon(pg[:, :3].reshape(BN, K, 3))
    ox, oy, oz, pidx = _ordk(cp2[:, :, 0], cp2[:, :, 1], cp2[:, :, 2], idxg)
    canon3 = jnp.stack([ox, oy, oz], axis=-1)
    if first:
        feat_full = canon3.reshape(BN, 3 * K)
    else:
        alig = gatherF(feattab, pidx.reshape(-1))
        feat_full = jnp.concatenate(
            [canon3, alig[:, :C].reshape(BN, K, C)], axis=-1).reshape(
                BN, K * (3 + C))
    xT = _mk_conv(O, K * 3 if first else K * (3 + C))(
        W, feat_full, g.reshape(O, 1), b.reshape(O, 1), sc)
    xTb = jnp.transpose(xT.reshape(O, B, N), (1, 0, 2))
    rows = jnp.transpose(xTb, (0, 2, 1)).reshape(BN, O)
    return xTb, rows


def kernel(x, W1, g1, b1, W2, g2, b2, W3, g3, b3, W4, g4, b4, W5, g5, b5,
           L1, g6, b6, L2, L2b, g7, b7, L3, L3b):
    sc = (1.0 / jnp.sqrt(1.0 + jnp.float32(1e-5))).reshape(1, 1)
    xt = jnp.transpose(x, (0, 2, 1))
    gcp2 = _canon(xt)
    gx, gy, gz = gcp2[:, :, 0], gcp2[:, :, 1], gcp2[:, :, 2]
    ox, oy, oz = _gord(gx[:, :, None], gy[:, :, None], gz[:, :, None],
                       gx[:, None, :], gy[:, None, :], gz[:, None, :])
    pts = jnp.concatenate([ox, oy, oz], axis=-1)
    ptsflat = pts.reshape(BN, 3)
    ptstab = jnp.pad(ptsflat, ((0, 0), (0, 13)))
    x0T = jnp.transpose(pts, (0, 2, 1))

    x1T, x1rows = _stage(x0T, ptstab, None, None, W1, g1, b1, sc, True, 64, 3)
    x2T, x2rows = _stage(x1T, ptstab, x1rows, _gather64, W2, g2, b2, sc,
                         False, 64, 64)
    x3T, x3rows = _stage(x2T, ptstab, x2rows, _gather64, W3, g3, b3, sc,
                         False, 128, 64)
    _, x4rows = _stage(x3T, ptstab, x3rows, _gather128, W4, g4, b4, sc,
                       False, 256, 128)

    A = jnp.concatenate([x1rows, x2rows, x3rows, x4rows], axis=-1)
    mx, mn = _embed(W5, A.reshape(B, N, 512), g5.reshape(1024, 1),
                    b5.reshape(1024, 1), sc)
    xp = jnp.concatenate([mx.reshape(B, 1024), mn.reshape(B, 1024)], axis=-1)
    return _head(xp, L1, g6.reshape(1, 512), b6.reshape(1, 512), L2,
                 L2b.reshape(1, 256), g7.reshape(1, 256), b7.reshape(1, 256),
                 L3, L3b.reshape(1, 40), sc)
